# speedup vs baseline: 7.9783x; 7.9783x over previous
import jax
import jax.numpy as jnp
from jax.experimental import pallas as pl

E = 8
D = 768
DFFN = 1536
S = 2048
BLK = 512


def _route_body(x_ref, rw_ref, wall_ref):
    x = x_ref[...]
    rw = rw_ref[...]
    logits = jax.lax.dot_general(x, rw, (((1,), (1,)), ((), ())),
                                 preferred_element_type=jnp.float32)
    m = jnp.max(logits, axis=1, keepdims=True)
    ex = jnp.exp(logits - m)
    p = ex / jnp.sum(ex, axis=1, keepdims=True)
    idx = jax.lax.broadcasted_iota(jnp.int32, p.shape, 1)
    m1 = jnp.max(p, axis=1, keepdims=True)
    a1 = jnp.min(jnp.where(p == m1, idx, E), axis=1, keepdims=True)
    pm = jnp.where(idx == a1, -1.0, p)
    m2 = jnp.max(pm, axis=1, keepdims=True)
    a2 = jnp.min(jnp.where(pm == m2, idx, E), axis=1, keepdims=True)
    s = m1 + m2
    wall = jnp.where(idx == a1, m1 / s, 0.0) + jnp.where(idx == a2, m2 / s, 0.0)
    wall_ref[...] = wall


def _moe_body(x_ref, w1_ref, w2_ref, wall_ref, o_ref):
    e = pl.program_id(1)
    x = x_ref[...]
    h = jnp.dot(x, w1_ref[...], preferred_element_type=jnp.float32)
    h = 0.5 * h * (1.0 + jax.lax.erf(h * 0.7071067811865476))
    o = jnp.dot(h, w2_ref[...], preferred_element_type=jnp.float32)
    wall = wall_ref[...]
    lane = jax.lax.broadcasted_iota(jnp.int32, wall.shape, 1)
    c = jnp.sum(jnp.where(lane == e, wall, 0.0), axis=1, keepdims=True)
    contrib = o * c

    @pl.when(e == 0)
    def _():
        o_ref[...] = contrib

    @pl.when(e > 0)
    def _():
        o_ref[...] += contrib


def kernel(x, router_w, w1, w2):
    xf = x.reshape(S, D)
    wall = pl.pallas_call(
        _route_body,
        out_shape=jax.ShapeDtypeStruct((S, E), jnp.float32),
    )(xf, router_w)
    y = pl.pallas_call(
        _moe_body,
        grid=(S // BLK, E),
        in_specs=[
            pl.BlockSpec((BLK, D), lambda i, e: (i, 0)),
            pl.BlockSpec((D, DFFN), lambda i, e: (0, e)),
            pl.BlockSpec((DFFN, D), lambda i, e: (e, 0)),
            pl.BlockSpec((BLK, E), lambda i, e: (i, 0)),
        ],
        out_specs=pl.BlockSpec((BLK, D), lambda i, e: (i, 0)),
        out_shape=jax.ShapeDtypeStruct((S, D), jnp.float32),
    )(xf, w1, w2, wall)
    return y.reshape(1, S, D)
